# baseline (speedup 1.0000x reference)
import functools

import jax
import jax.numpy as jnp
from jax import lax
from jax.experimental import pallas as pl
from jax.experimental.pallas import tpu as pltpu
from jax.experimental.pallas import tpu_sc as plsc

CUTOFF = 3.5
R2 = CUTOFF * CUTOFF
K_NEIGH = 16
TA_DEG = 0.6081734479693927 * 180.0
DELTA_THETA = 12.0
N_ATOMS = 20000
M_QUERY = 4096
LANES = 16
N_CHUNKS = N_ATOMS // LANES
N_WORKERS = 32
Q_PER_W = M_QUERY // N_WORKERS
CAP = 64
BIG = 1.0e30
SENTINEL = 1000.0


def _sc_neighbors(xs, ys, zs, qidx):
    mesh = plsc.VectorSubcoreMesh(core_axis_name="c", subcore_axis_name="s")
    out_sd = jax.ShapeDtypeStruct((M_QUERY * K_NEIGH,), jnp.float32)

    @functools.partial(
        pl.kernel,
        mesh=mesh,
        out_type=(out_sd, out_sd, out_sd),
        compiler_params=pltpu.CompilerParams(needs_layout_passes=False),
        scratch_types=[
            pltpu.VMEM((N_ATOMS,), jnp.float32),
            pltpu.VMEM((N_ATOMS,), jnp.float32),
            pltpu.VMEM((N_ATOMS,), jnp.float32),
            pltpu.VMEM((N_ATOMS,), jnp.float32),
            pltpu.VMEM((Q_PER_W,), jnp.int32),
            pltpu.VMEM((CAP,), jnp.float32),
            pltpu.VMEM((CAP,), jnp.int32),
            pltpu.VMEM((Q_PER_W * K_NEIGH,), jnp.int32),
            pltpu.VMEM((Q_PER_W * K_NEIGH,), jnp.float32),
            pltpu.VMEM((Q_PER_W * K_NEIGH,), jnp.float32),
            pltpu.VMEM((Q_PER_W * K_NEIGH,), jnp.float32),
        ],
    )
    def k(xs_hbm, ys_hbm, zs_hbm, qidx_hbm, ox_hbm, oy_hbm, oz_hbm,
          xb, yb, zb, kn_v, qidx_v, cand_ns, cand_ix, bidx_v,
          outx_v, outy_v, outz_v):
        wid = lax.axis_index("s") * 2 + lax.axis_index("c")
        qbase = wid * Q_PER_W

        pltpu.sync_copy(xs_hbm, xb)
        pltpu.sync_copy(ys_hbm, yb)
        pltpu.sync_copy(zs_hbm, zb)
        pltpu.sync_copy(qidx_hbm.at[pl.ds(qbase, Q_PER_W)], qidx_v)

        iota16 = lax.iota(jnp.int32, LANES)

        def bf16r(v):
            u = plsc.bitcast(v, jnp.uint32)
            lsb = (u >> jnp.uint32(16)) & jnp.uint32(1)
            r = (u + jnp.uint32(0x7FFF) + lsb) & jnp.uint32(0xFFFF0000)
            return plsc.bitcast(r, jnp.float32)

        def prep(c, _):
            sl = pl.ds(c * LANES, LANES)
            x = xb[sl]
            y = yb[sl]
            z = zb[sl]
            kn_v[sl] = x * x + y * y + z * z
            xb[sl] = bf16r(x)
            yb[sl] = bf16r(y)
            zb[sl] = bf16r(z)
            return 0

        lax.fori_loop(0, N_CHUNKS, prep, 0)

        def merge16(cnt):
            bd = jnp.full((LANES,), BIG, jnp.float32)
            bi = jnp.zeros((LANES,), jnp.int32)
            for jj in range(CAP // LANES):
                base = jj * LANES
                dv = cand_ns[pl.ds(base, LANES)]
                iv = cand_ix[pl.ds(base, LANES)]
                dv = jnp.where((base + iota16) < cnt, dv, BIG)
                dv, iv = plsc.sort_key_val(dv, iv)
                rd = lax.rev(bd, (0,))
                ri = lax.rev(bi, (0,))
                take = dv <= rd
                bd = jnp.where(take, dv, rd)
                bi = jnp.where(take, iv, ri)
                bd, bi = plsc.sort_key_val(bd, bi)
            return bd, bi

        def per_query(q, _):
            qsplat = jnp.full((LANES,), 0, jnp.int32) + q
            qi = plsc.load_gather(qidx_v, [qsplat])
            qx = plsc.load_gather(xb, [qi])
            qy = plsc.load_gather(yb, [qi])
            qz = plsc.load_gather(zb, [qi])
            qn = plsc.load_gather(kn_v, [qi])

            def chunk_body(c, carry):
                cntv, idxv, thr = carry
                sl = pl.ds(c * LANES, LANES)
                prod = qx * xb[sl] + qy * yb[sl] + qz * zb[sl]
                ns = (qn + kn_v[sl]) - 2.0 * prod
                m = (ns <= thr) & (idxv != qi)

                def do_append(cv, th):
                    ones = jnp.where(m, 1, 0).astype(jnp.int32)
                    pos = cv + plsc.cumsum(ones) - 1
                    pos = jnp.minimum(pos, CAP - 1)
                    plsc.store_scatter(cand_ns, [pos], ns, mask=m)
                    plsc.store_scatter(cand_ix, [pos], idxv, mask=m)
                    cv = cv + plsc.all_reduce_population_count(m)

                    def consolidate(_cv, _th):
                        bd, bi = merge16(jnp.max(_cv))
                        cand_ns[pl.ds(0, LANES)] = bd
                        cand_ix[pl.ds(0, LANES)] = bi
                        return (jnp.full((LANES,), LANES, jnp.int32),
                                jnp.max(bd))

                    return lax.cond(jnp.max(cv) > CAP - LANES,
                                    consolidate, lambda a, b: (a, b), cv, th)

                cntv, thr = lax.cond(jnp.any(m), do_append,
                                     lambda a, b: (a, b), cntv, thr)
                return (cntv, idxv + LANES, thr)

            cnt0 = jnp.zeros((LANES,), jnp.int32)
            cntv, _, _ = lax.fori_loop(
                0, N_CHUNKS, chunk_body, (cnt0, iota16, jnp.float32(BIG)))
            _, bi = merge16(jnp.max(cntv))
            bidx_v[pl.ds(q * LANES, LANES)] = bi
            return 0

        lax.fori_loop(0, Q_PER_W, per_query, 0)

        pltpu.sync_copy(xs_hbm, xb)
        pltpu.sync_copy(ys_hbm, yb)
        pltpu.sync_copy(zs_hbm, zb)

        def emit(q, _):
            qsplat = jnp.full((LANES,), 0, jnp.int32) + q
            qi = plsc.load_gather(qidx_v, [qsplat])
            qx = plsc.load_gather(xb, [qi])
            qy = plsc.load_gather(yb, [qi])
            qz = plsc.load_gather(zb, [qi])
            osl = pl.ds(q * LANES, LANES)
            bi = bidx_v[osl]
            outx_v[osl] = plsc.load_gather(xb, [bi]) - qx
            outy_v[osl] = plsc.load_gather(yb, [bi]) - qy
            outz_v[osl] = plsc.load_gather(zb, [bi]) - qz
            return 0

        lax.fori_loop(0, Q_PER_W, emit, 0)

        obase = qbase * K_NEIGH
        osl = pl.ds(obase, Q_PER_W * K_NEIGH)
        pltpu.sync_copy(outx_v, ox_hbm.at[osl])
        pltpu.sync_copy(outy_v, oy_hbm.at[osl])
        pltpu.sync_copy(outz_v, oz_hbm.at[osl])

    return k(xs, ys, zs, qidx)


_ACOS_C = (1.5707963050, -0.2145988016, 0.0889789874, -0.0501743046,
           0.0308918810, -0.0170881256, 0.0066700901, -0.0012624911)
_PI = 3.14159265358979


def _geom_body(vx_ref, vy_ref, vz_ref, o_ref):
    vx = vx_ref[...]
    vy = vy_ref[...]
    vz = vz_ref[...]
    mb = vx.shape[1]
    d2 = vx * vx + vy * vy + vz * vz
    dist = jnp.sqrt(jnp.maximum(d2, 1e-12))
    vm = (dist <= CUTOFF).astype(jnp.float32)
    inv = vm / dist
    ux = vx * inv
    uy = vy * inv
    uz = vz * inv

    rows = lax.broadcasted_iota(jnp.int32, (K_NEIGH, mb), 0)
    acc_n = jnp.zeros((K_NEIGH, mb), jnp.float32)
    acc_g = jnp.zeros((K_NEIGH, mb), jnp.float32)
    acc_4 = jnp.zeros((K_NEIGH, mb), jnp.float32)
    acc_6 = jnp.zeros((K_NEIGH, mb), jnp.float32)
    for j in range(K_NEIGH):
        cosg = (ux[j:j + 1, :] * ux + uy[j:j + 1, :] * uy
                + uz[j:j + 1, :] * uz)
        cosg = jnp.clip(cosg, -1.0, 1.0)
        pv = vm[j:j + 1, :] * vm
        pjk = jnp.where(rows > j, pv, 0.0)
        acc_n = acc_n + pjk

        sc = jnp.where(pjk > 0, jnp.clip(cosg, -0.999999, 0.999999), 0.0)
        ax = jnp.abs(sc)
        s = jnp.sqrt(jnp.maximum(1.0 - ax, 0.0))
        p = _ACOS_C[7]
        for c in reversed(_ACOS_C[:7]):
            p = p * ax + c
        acos = jnp.where(sc < 0.0, _PI - s * p, s * p)
        theta = acos * (180.0 / _PI)
        z = (theta - TA_DEG) * (1.0 / DELTA_THETA)
        acc_g = acc_g + jnp.exp(-0.5 * z * z) * pjk

        x2 = cosg * cosg
        x4 = x2 * x2
        x6 = x4 * x2
        acc_4 = acc_4 + ((35.0 * x4 - 30.0 * x2 + 3.0) * 0.125) * pv
        acc_6 = acc_6 + ((231.0 * x6 - 315.0 * x4 + 105.0 * x2 - 5.0)
                         * 0.0625) * pv

    b = jnp.concatenate([vm, acc_n, acc_g, acc_4, acc_6], axis=0)
    wr = lax.broadcasted_iota(jnp.int32, (8, 5 * K_NEIGH), 0)
    wc = lax.broadcasted_iota(jnp.int32, (8, 5 * K_NEIGH), 1)
    w = jnp.where(wc // K_NEIGH == wr, 1.0, 0.0).astype(jnp.float32)
    sums = jax.lax.dot_general(w, b, (((1,), (0,)), ((), ())),
                               preferred_element_type=jnp.float32)
    cn = sums[0:1, :]
    npairs = sums[1:2, :]
    gs = sums[2:3, :]
    s4 = sums[3:4, :]
    s6 = sums[4:5, :]
    tet = gs / jnp.maximum(npairs, 1.0)
    dm = jnp.maximum(cn, 1.0)
    q4 = jnp.sqrt(jnp.maximum(s4, 1e-12)) / dm
    q6 = jnp.sqrt(jnp.maximum(s6, 1e-12)) / dm
    o_ref[0:1, :] = cn
    o_ref[1:2, :] = tet
    o_ref[2:3, :] = q4
    o_ref[3:4, :] = q6
    o_ref[4:8, :] = jnp.zeros((4, mb), jnp.float32)


def _tc_geometry(vx, vy, vz):
    mb = 256
    grid = (M_QUERY // mb,)
    spec = pl.BlockSpec((LANES, mb), lambda i: (0, i))
    out = pl.pallas_call(
        _geom_body,
        grid=grid,
        in_specs=[spec, spec, spec],
        out_specs=pl.BlockSpec((8, mb), lambda i: (0, i)),
        out_shape=jax.ShapeDtypeStruct((8, M_QUERY), jnp.float32),
    )(vx, vy, vz)
    return out[:4]


@jax.jit
def kernel(positions, atom_indices):
    pos = positions.astype(jnp.float32)
    xs = pos[:, 0]
    ys = pos[:, 1]
    zs = pos[:, 2]
    qidx = atom_indices.astype(jnp.int32)
    ox, oy, oz = _sc_neighbors(xs, ys, zs, qidx)
    vx = ox.reshape(M_QUERY, K_NEIGH).T
    vy = oy.reshape(M_QUERY, K_NEIGH).T
    vz = oz.reshape(M_QUERY, K_NEIGH).T
    return _tc_geometry(vx, vy, vz)


# speedup vs baseline: 3.8620x; 3.8620x over previous
import functools

import jax
import jax.numpy as jnp
from jax import lax
from jax.experimental import pallas as pl
from jax.experimental.pallas import tpu as pltpu
from jax.experimental.pallas import tpu_sc as plsc

CUTOFF = 3.5
R2 = CUTOFF * CUTOFF
K_NEIGH = 16
TA_DEG = 0.6081734479693927 * 180.0
DELTA_THETA = 12.0
N_ATOMS = 20000
M_QUERY = 4096
LANES = 16
N_CHUNKS = N_ATOMS // LANES
N_WORKERS = 32
Q_PER_W = M_QUERY // N_WORKERS
GROUP = 5
CAP = 128
BIG = 1.0e30
SENTINEL = 1000.0


def _sc_neighbors(xs, ys, zs, qidx):
    mesh = plsc.VectorSubcoreMesh(core_axis_name="c", subcore_axis_name="s")
    out_sd = jax.ShapeDtypeStruct((M_QUERY * K_NEIGH,), jnp.float32)

    @functools.partial(
        pl.kernel,
        mesh=mesh,
        out_type=(out_sd, out_sd, out_sd),
        compiler_params=pltpu.CompilerParams(needs_layout_passes=False),
        scratch_types=[
            pltpu.VMEM((N_ATOMS,), jnp.float32),
            pltpu.VMEM((N_ATOMS,), jnp.float32),
            pltpu.VMEM((N_ATOMS,), jnp.float32),
            pltpu.VMEM((N_ATOMS,), jnp.float32),
            pltpu.VMEM((Q_PER_W,), jnp.int32),
            pltpu.VMEM((CAP,), jnp.float32),
            pltpu.VMEM((CAP,), jnp.int32),
            pltpu.VMEM((Q_PER_W * K_NEIGH,), jnp.int32),
            pltpu.VMEM((Q_PER_W * K_NEIGH,), jnp.float32),
            pltpu.VMEM((Q_PER_W * K_NEIGH,), jnp.float32),
            pltpu.VMEM((Q_PER_W * K_NEIGH,), jnp.float32),
        ],
    )
    def k(xs_hbm, ys_hbm, zs_hbm, qidx_hbm, ox_hbm, oy_hbm, oz_hbm,
          xb, yb, zb, kn_v, qidx_v, cand_ns, cand_ix, bidx_v,
          outx_v, outy_v, outz_v):
        wid = lax.axis_index("s") * 2 + lax.axis_index("c")
        qbase = wid * Q_PER_W

        pltpu.sync_copy(xs_hbm, xb)
        pltpu.sync_copy(ys_hbm, yb)
        pltpu.sync_copy(zs_hbm, zb)
        pltpu.sync_copy(qidx_hbm.at[pl.ds(qbase, Q_PER_W)], qidx_v)

        iota16 = lax.iota(jnp.int32, LANES)

        def bf16r(v):
            u = plsc.bitcast(v, jnp.uint32)
            lsb = (u >> jnp.uint32(16)) & jnp.uint32(1)
            r = (u + jnp.uint32(0x7FFF) + lsb) & jnp.uint32(0xFFFF0000)
            return plsc.bitcast(r, jnp.float32)

        def prep(c, _):
            sl = pl.ds(c * LANES, LANES)
            x = xb[sl]
            y = yb[sl]
            z = zb[sl]
            kn_v[sl] = x * x + y * y + z * z
            xb[sl] = bf16r(x)
            yb[sl] = bf16r(y)
            zb[sl] = bf16r(z)
            return 0

        lax.fori_loop(0, N_CHUNKS, prep, 0)

        def merge16(cnt):
            bd = jnp.full((LANES,), BIG, jnp.float32)
            bi = jnp.zeros((LANES,), jnp.int32)
            for jj in range(CAP // LANES):
                base = jj * LANES
                dv = cand_ns[pl.ds(base, LANES)]
                iv = cand_ix[pl.ds(base, LANES)]
                dv = jnp.where((base + iota16) < cnt, dv, BIG)
                dv, iv = plsc.sort_key_val(dv, iv)
                rd = lax.rev(bd, (0,))
                ri = lax.rev(bi, (0,))
                take = dv <= rd
                bd = jnp.where(take, dv, rd)
                bi = jnp.where(take, iv, ri)
                bd, bi = plsc.sort_key_val(bd, bi)
            return bd, bi

        def per_query(q, _):
            qsplat = jnp.full((LANES,), 0, jnp.int32) + q
            qi = plsc.load_gather(qidx_v, [qsplat])
            qx = plsc.load_gather(xb, [qi])
            qy = plsc.load_gather(yb, [qi])
            qz = plsc.load_gather(zb, [qi])
            qn = plsc.load_gather(kn_v, [qi])

            def group_body(g, carry):
                cntv, thr = carry
                base = g * (LANES * GROUP)
                ms = []
                nss = []
                idxs = []
                for u in range(GROUP):
                    off = base + u * LANES
                    sl = pl.ds(off, LANES)
                    idxv = off + iota16
                    prod = qx * xb[sl] + qy * yb[sl] + qz * zb[sl]
                    ns = (qn + kn_v[sl]) - 2.0 * prod
                    ms.append((ns <= thr) & (idxv != qi))
                    nss.append(ns)
                    idxs.append(idxv)
                mor = ms[0]
                for u in range(1, GROUP):
                    mor = mor | ms[u]

                def do_append(cv, th):
                    for u in range(GROUP):
                        ones = jnp.where(ms[u], 1, 0).astype(jnp.int32)
                        pos = cv + plsc.cumsum(ones) - 1
                        pos = jnp.minimum(pos, CAP - 1)
                        plsc.store_scatter(cand_ns, [pos], nss[u], mask=ms[u])
                        plsc.store_scatter(cand_ix, [pos], idxs[u], mask=ms[u])
                        cv = cv + plsc.all_reduce_population_count(ms[u])

                    def consolidate(_cv, _th):
                        bd, bi = merge16(jnp.max(_cv))
                        cand_ns[pl.ds(0, LANES)] = bd
                        cand_ix[pl.ds(0, LANES)] = bi
                        return (jnp.full((LANES,), LANES, jnp.int32),
                                jnp.max(bd))

                    return lax.cond(jnp.max(cv) > CAP - LANES * GROUP,
                                    consolidate, lambda a, b: (a, b), cv, th)

                cntv, thr = lax.cond(jnp.any(mor), do_append,
                                     lambda a, b: (a, b), cntv, thr)
                return (cntv, thr)

            cnt0 = jnp.zeros((LANES,), jnp.int32)
            cntv, _ = lax.fori_loop(
                0, N_CHUNKS // GROUP, group_body, (cnt0, jnp.float32(BIG)))
            _, bi = merge16(jnp.max(cntv))
            bidx_v[pl.ds(q * LANES, LANES)] = bi
            return 0

        lax.fori_loop(0, Q_PER_W, per_query, 0)

        pltpu.sync_copy(xs_hbm, xb)
        pltpu.sync_copy(ys_hbm, yb)
        pltpu.sync_copy(zs_hbm, zb)

        def emit(q, _):
            qsplat = jnp.full((LANES,), 0, jnp.int32) + q
            qi = plsc.load_gather(qidx_v, [qsplat])
            qx = plsc.load_gather(xb, [qi])
            qy = plsc.load_gather(yb, [qi])
            qz = plsc.load_gather(zb, [qi])
            osl = pl.ds(q * LANES, LANES)
            bi = bidx_v[osl]
            outx_v[osl] = plsc.load_gather(xb, [bi]) - qx
            outy_v[osl] = plsc.load_gather(yb, [bi]) - qy
            outz_v[osl] = plsc.load_gather(zb, [bi]) - qz
            return 0

        lax.fori_loop(0, Q_PER_W, emit, 0)

        obase = qbase * K_NEIGH
        osl = pl.ds(obase, Q_PER_W * K_NEIGH)
        pltpu.sync_copy(outx_v, ox_hbm.at[osl])
        pltpu.sync_copy(outy_v, oy_hbm.at[osl])
        pltpu.sync_copy(outz_v, oz_hbm.at[osl])

    return k(xs, ys, zs, qidx)


_ACOS_C = (1.5707963050, -0.2145988016, 0.0889789874, -0.0501743046,
           0.0308918810, -0.0170881256, 0.0066700901, -0.0012624911)
_PI = 3.14159265358979


def _geom_body(vx_ref, vy_ref, vz_ref, o_ref):
    vx = vx_ref[...]
    vy = vy_ref[...]
    vz = vz_ref[...]
    mb = vx.shape[1]
    d2 = vx * vx + vy * vy + vz * vz
    dist = jnp.sqrt(jnp.maximum(d2, 1e-12))
    vm = (dist <= CUTOFF).astype(jnp.float32)
    inv = vm / dist
    ux = vx * inv
    uy = vy * inv
    uz = vz * inv

    rows = lax.broadcasted_iota(jnp.int32, (K_NEIGH, mb), 0)
    acc_n = jnp.zeros((K_NEIGH, mb), jnp.float32)
    acc_g = jnp.zeros((K_NEIGH, mb), jnp.float32)
    acc_4 = jnp.zeros((K_NEIGH, mb), jnp.float32)
    acc_6 = jnp.zeros((K_NEIGH, mb), jnp.float32)
    for j in range(K_NEIGH):
        cosg = (ux[j:j + 1, :] * ux + uy[j:j + 1, :] * uy
                + uz[j:j + 1, :] * uz)
        cosg = jnp.clip(cosg, -1.0, 1.0)
        pv = vm[j:j + 1, :] * vm
        pjk = jnp.where(rows > j, pv, 0.0)
        acc_n = acc_n + pjk

        sc = jnp.where(pjk > 0, jnp.clip(cosg, -0.999999, 0.999999), 0.0)
        ax = jnp.abs(sc)
        s = jnp.sqrt(jnp.maximum(1.0 - ax, 0.0))
        p = _ACOS_C[7]
        for c in reversed(_ACOS_C[:7]):
            p = p * ax + c
        acos = jnp.where(sc < 0.0, _PI - s * p, s * p)
        theta = acos * (180.0 / _PI)
        z = (theta - TA_DEG) * (1.0 / DELTA_THETA)
        acc_g = acc_g + jnp.exp(-0.5 * z * z) * pjk

        x2 = cosg * cosg
        x4 = x2 * x2
        x6 = x4 * x2
        acc_4 = acc_4 + ((35.0 * x4 - 30.0 * x2 + 3.0) * 0.125) * pv
        acc_6 = acc_6 + ((231.0 * x6 - 315.0 * x4 + 105.0 * x2 - 5.0)
                         * 0.0625) * pv

    b = jnp.concatenate([vm, acc_n, acc_g, acc_4, acc_6], axis=0)
    wr = lax.broadcasted_iota(jnp.int32, (8, 5 * K_NEIGH), 0)
    wc = lax.broadcasted_iota(jnp.int32, (8, 5 * K_NEIGH), 1)
    w = jnp.where(wc // K_NEIGH == wr, 1.0, 0.0).astype(jnp.float32)
    sums = jax.lax.dot_general(w, b, (((1,), (0,)), ((), ())),
                               preferred_element_type=jnp.float32)
    cn = sums[0:1, :]
    npairs = sums[1:2, :]
    gs = sums[2:3, :]
    s4 = sums[3:4, :]
    s6 = sums[4:5, :]
    tet = gs / jnp.maximum(npairs, 1.0)
    dm = jnp.maximum(cn, 1.0)
    q4 = jnp.sqrt(jnp.maximum(s4, 1e-12)) / dm
    q6 = jnp.sqrt(jnp.maximum(s6, 1e-12)) / dm
    o_ref[0:1, :] = cn
    o_ref[1:2, :] = tet
    o_ref[2:3, :] = q4
    o_ref[3:4, :] = q6
    o_ref[4:8, :] = jnp.zeros((4, mb), jnp.float32)


def _tc_geometry(vx, vy, vz):
    mb = 256
    grid = (M_QUERY // mb,)
    spec = pl.BlockSpec((LANES, mb), lambda i: (0, i))
    out = pl.pallas_call(
        _geom_body,
        grid=grid,
        in_specs=[spec, spec, spec],
        out_specs=pl.BlockSpec((8, mb), lambda i: (0, i)),
        out_shape=jax.ShapeDtypeStruct((8, M_QUERY), jnp.float32),
    )(vx, vy, vz)
    return out[:4]


@jax.jit
def kernel(positions, atom_indices):
    pos = positions.astype(jnp.float32)
    xs = pos[:, 0]
    ys = pos[:, 1]
    zs = pos[:, 2]
    qidx = atom_indices.astype(jnp.int32)
    ox, oy, oz = _sc_neighbors(xs, ys, zs, qidx)
    vx = ox.reshape(M_QUERY, K_NEIGH).T
    vy = oy.reshape(M_QUERY, K_NEIGH).T
    vz = oz.reshape(M_QUERY, K_NEIGH).T
    return _tc_geometry(vx, vy, vz)


# speedup vs baseline: 5.6232x; 1.4560x over previous
import functools

import jax
import jax.numpy as jnp
from jax import lax
from jax.experimental import pallas as pl
from jax.experimental.pallas import tpu as pltpu
from jax.experimental.pallas import tpu_sc as plsc

CUTOFF = 3.5
R2 = CUTOFF * CUTOFF
K_NEIGH = 16
TA_DEG = 0.6081734479693927 * 180.0
DELTA_THETA = 12.0
N_ATOMS = 20000
M_QUERY = 4096
LANES = 16
N_CHUNKS = N_ATOMS // LANES
N_WORKERS = 32
Q_PER_W = M_QUERY // N_WORKERS
GROUP = 10
CAP = 256
BIG = 1.0e30
SENTINEL = 1000.0


def _sc_neighbors(xs, ys, zs, qidx):
    mesh = plsc.VectorSubcoreMesh(core_axis_name="c", subcore_axis_name="s")
    out_sd = jax.ShapeDtypeStruct((M_QUERY * K_NEIGH,), jnp.float32)

    @functools.partial(
        pl.kernel,
        mesh=mesh,
        out_type=(out_sd, out_sd, out_sd),
        compiler_params=pltpu.CompilerParams(needs_layout_passes=False),
        scratch_types=[
            pltpu.VMEM((N_ATOMS,), jnp.float32),
            pltpu.VMEM((N_ATOMS,), jnp.float32),
            pltpu.VMEM((N_ATOMS,), jnp.float32),
            pltpu.VMEM((N_ATOMS,), jnp.float32),
            pltpu.VMEM((Q_PER_W,), jnp.int32),
            pltpu.VMEM((CAP,), jnp.float32),
            pltpu.VMEM((CAP,), jnp.int32),
            pltpu.VMEM((Q_PER_W * K_NEIGH,), jnp.int32),
            pltpu.VMEM((Q_PER_W * K_NEIGH,), jnp.float32),
            pltpu.VMEM((Q_PER_W * K_NEIGH,), jnp.float32),
            pltpu.VMEM((Q_PER_W * K_NEIGH,), jnp.float32),
        ],
    )
    def k(xs_hbm, ys_hbm, zs_hbm, qidx_hbm, ox_hbm, oy_hbm, oz_hbm,
          xb, yb, zb, kn_v, qidx_v, cand_ns, cand_ix, bidx_v,
          outx_v, outy_v, outz_v):
        wid = lax.axis_index("s") * 2 + lax.axis_index("c")
        qbase = wid * Q_PER_W

        pltpu.sync_copy(xs_hbm, xb)
        pltpu.sync_copy(ys_hbm, yb)
        pltpu.sync_copy(zs_hbm, zb)
        pltpu.sync_copy(qidx_hbm.at[pl.ds(qbase, Q_PER_W)], qidx_v)

        iota16 = lax.iota(jnp.int32, LANES)

        def bf16r(v):
            u = plsc.bitcast(v, jnp.uint32)
            lsb = (u >> jnp.uint32(16)) & jnp.uint32(1)
            r = (u + jnp.uint32(0x7FFF) + lsb) & jnp.uint32(0xFFFF0000)
            return plsc.bitcast(r, jnp.float32)

        def prep(c, _):
            sl = pl.ds(c * LANES, LANES)
            x = xb[sl]
            y = yb[sl]
            z = zb[sl]
            kn_v[sl] = x * x + y * y + z * z
            xb[sl] = bf16r(x)
            yb[sl] = bf16r(y)
            zb[sl] = bf16r(z)
            return 0

        lax.fori_loop(0, N_CHUNKS, prep, 0)

        def merge16(cnt):
            bd = jnp.full((LANES,), BIG, jnp.float32)
            bi = jnp.zeros((LANES,), jnp.int32)
            for jj in range(CAP // LANES):
                base = jj * LANES
                dv = cand_ns[pl.ds(base, LANES)]
                iv = cand_ix[pl.ds(base, LANES)]
                dv = jnp.where((base + iota16) < cnt, dv, BIG)
                dv, iv = plsc.sort_key_val(dv, iv)
                rd = lax.rev(bd, (0,))
                ri = lax.rev(bi, (0,))
                take = dv <= rd
                bd = jnp.where(take, dv, rd)
                bi = jnp.where(take, iv, ri)
                bd, bi = plsc.sort_key_val(bd, bi)
            return bd, bi

        def per_query(q, _):
            qsplat = jnp.full((LANES,), 0, jnp.int32) + q
            qi = plsc.load_gather(qidx_v, [qsplat])
            qx = plsc.load_gather(xb, [qi])
            qy = plsc.load_gather(yb, [qi])
            qz = plsc.load_gather(zb, [qi])
            qn = plsc.load_gather(kn_v, [qi])

            def group_body(g, carry):
                cntv, thr = carry
                base = g * (LANES * GROUP)
                ms = []
                nss = []
                idxs = []
                for u in range(GROUP):
                    off = base + u * LANES
                    sl = pl.ds(off, LANES)
                    idxv = off + iota16
                    prod = qx * xb[sl] + qy * yb[sl] + qz * zb[sl]
                    ns = (qn + kn_v[sl]) - 2.0 * prod
                    ms.append((ns <= thr) & (idxv != qi))
                    nss.append(ns)
                    idxs.append(idxv)
                mor = ms[0]
                for u in range(1, GROUP):
                    mor = mor | ms[u]

                def do_append(cv, th):
                    for u in range(GROUP):
                        ones = jnp.where(ms[u], 1, 0).astype(jnp.int32)
                        pos = cv + plsc.cumsum(ones) - 1
                        pos = jnp.minimum(pos, CAP - 1)
                        plsc.store_scatter(cand_ns, [pos], nss[u], mask=ms[u])
                        plsc.store_scatter(cand_ix, [pos], idxs[u], mask=ms[u])
                        cv = cv + plsc.all_reduce_population_count(ms[u])

                    def consolidate(_cv, _th):
                        bd, bi = merge16(jnp.max(_cv))
                        cand_ns[pl.ds(0, LANES)] = bd
                        cand_ix[pl.ds(0, LANES)] = bi
                        return (jnp.full((LANES,), LANES, jnp.int32),
                                jnp.max(bd))

                    return lax.cond(jnp.max(cv) > CAP - LANES * GROUP,
                                    consolidate, lambda a, b: (a, b), cv, th)

                cntv, thr = lax.cond(jnp.any(mor), do_append,
                                     lambda a, b: (a, b), cntv, thr)
                return (cntv, thr)

            cnt0 = jnp.zeros((LANES,), jnp.int32)
            cntv, _ = lax.fori_loop(
                0, N_CHUNKS // GROUP, group_body, (cnt0, jnp.float32(BIG)))
            _, bi = merge16(jnp.max(cntv))
            bidx_v[pl.ds(q * LANES, LANES)] = bi
            return 0

        lax.fori_loop(0, Q_PER_W, per_query, 0)

        pltpu.sync_copy(xs_hbm, xb)
        pltpu.sync_copy(ys_hbm, yb)
        pltpu.sync_copy(zs_hbm, zb)

        def emit(q, _):
            qsplat = jnp.full((LANES,), 0, jnp.int32) + q
            qi = plsc.load_gather(qidx_v, [qsplat])
            qx = plsc.load_gather(xb, [qi])
            qy = plsc.load_gather(yb, [qi])
            qz = plsc.load_gather(zb, [qi])
            osl = pl.ds(q * LANES, LANES)
            bi = bidx_v[osl]
            outx_v[osl] = plsc.load_gather(xb, [bi]) - qx
            outy_v[osl] = plsc.load_gather(yb, [bi]) - qy
            outz_v[osl] = plsc.load_gather(zb, [bi]) - qz
            return 0

        lax.fori_loop(0, Q_PER_W, emit, 0)

        obase = qbase * K_NEIGH
        osl = pl.ds(obase, Q_PER_W * K_NEIGH)
        pltpu.sync_copy(outx_v, ox_hbm.at[osl])
        pltpu.sync_copy(outy_v, oy_hbm.at[osl])
        pltpu.sync_copy(outz_v, oz_hbm.at[osl])

    return k(xs, ys, zs, qidx)


_ACOS_C = (1.5707963050, -0.2145988016, 0.0889789874, -0.0501743046,
           0.0308918810, -0.0170881256, 0.0066700901, -0.0012624911)
_PI = 3.14159265358979


def _geom_body(vx_ref, vy_ref, vz_ref, o_ref):
    vx = vx_ref[...]
    vy = vy_ref[...]
    vz = vz_ref[...]
    mb = vx.shape[1]
    d2 = vx * vx + vy * vy + vz * vz
    dist = jnp.sqrt(jnp.maximum(d2, 1e-12))
    vm = (dist <= CUTOFF).astype(jnp.float32)
    inv = vm / dist
    ux = vx * inv
    uy = vy * inv
    uz = vz * inv

    rows = lax.broadcasted_iota(jnp.int32, (K_NEIGH, mb), 0)
    acc_n = jnp.zeros((K_NEIGH, mb), jnp.float32)
    acc_g = jnp.zeros((K_NEIGH, mb), jnp.float32)
    acc_4 = jnp.zeros((K_NEIGH, mb), jnp.float32)
    acc_6 = jnp.zeros((K_NEIGH, mb), jnp.float32)
    for j in range(K_NEIGH):
        cosg = (ux[j:j + 1, :] * ux + uy[j:j + 1, :] * uy
                + uz[j:j + 1, :] * uz)
        cosg = jnp.clip(cosg, -1.0, 1.0)
        pv = vm[j:j + 1, :] * vm
        pjk = jnp.where(rows > j, pv, 0.0)
        acc_n = acc_n + pjk

        sc = jnp.where(pjk > 0, jnp.clip(cosg, -0.999999, 0.999999), 0.0)
        ax = jnp.abs(sc)
        s = jnp.sqrt(jnp.maximum(1.0 - ax, 0.0))
        p = _ACOS_C[7]
        for c in reversed(_ACOS_C[:7]):
            p = p * ax + c
        acos = jnp.where(sc < 0.0, _PI - s * p, s * p)
        theta = acos * (180.0 / _PI)
        z = (theta - TA_DEG) * (1.0 / DELTA_THETA)
        acc_g = acc_g + jnp.exp(-0.5 * z * z) * pjk

        x2 = cosg * cosg
        x4 = x2 * x2
        x6 = x4 * x2
        acc_4 = acc_4 + ((35.0 * x4 - 30.0 * x2 + 3.0) * 0.125) * pv
        acc_6 = acc_6 + ((231.0 * x6 - 315.0 * x4 + 105.0 * x2 - 5.0)
                         * 0.0625) * pv

    b = jnp.concatenate([vm, acc_n, acc_g, acc_4, acc_6], axis=0)
    wr = lax.broadcasted_iota(jnp.int32, (8, 5 * K_NEIGH), 0)
    wc = lax.broadcasted_iota(jnp.int32, (8, 5 * K_NEIGH), 1)
    w = jnp.where(wc // K_NEIGH == wr, 1.0, 0.0).astype(jnp.float32)
    sums = jax.lax.dot_general(w, b, (((1,), (0,)), ((), ())),
                               preferred_element_type=jnp.float32)
    cn = sums[0:1, :]
    npairs = sums[1:2, :]
    gs = sums[2:3, :]
    s4 = sums[3:4, :]
    s6 = sums[4:5, :]
    tet = gs / jnp.maximum(npairs, 1.0)
    dm = jnp.maximum(cn, 1.0)
    q4 = jnp.sqrt(jnp.maximum(s4, 1e-12)) / dm
    q6 = jnp.sqrt(jnp.maximum(s6, 1e-12)) / dm
    o_ref[0:1, :] = cn
    o_ref[1:2, :] = tet
    o_ref[2:3, :] = q4
    o_ref[3:4, :] = q6
    o_ref[4:8, :] = jnp.zeros((4, mb), jnp.float32)


def _tc_geometry(vx, vy, vz):
    mb = 256
    grid = (M_QUERY // mb,)
    spec = pl.BlockSpec((LANES, mb), lambda i: (0, i))
    out = pl.pallas_call(
        _geom_body,
        grid=grid,
        in_specs=[spec, spec, spec],
        out_specs=pl.BlockSpec((8, mb), lambda i: (0, i)),
        out_shape=jax.ShapeDtypeStruct((8, M_QUERY), jnp.float32),
    )(vx, vy, vz)
    return out[:4]


@jax.jit
def kernel(positions, atom_indices):
    pos = positions.astype(jnp.float32)
    xs = pos[:, 0]
    ys = pos[:, 1]
    zs = pos[:, 2]
    qidx = atom_indices.astype(jnp.int32)
    ox, oy, oz = _sc_neighbors(xs, ys, zs, qidx)
    vx = ox.reshape(M_QUERY, K_NEIGH).T
    vy = oy.reshape(M_QUERY, K_NEIGH).T
    vz = oz.reshape(M_QUERY, K_NEIGH).T
    return _tc_geometry(vx, vy, vz)
